# baseline (speedup 1.0000x reference)
import jax
import jax.numpy as jnp
from jax.experimental import pallas as pl
from jax.experimental.pallas import tpu as pltpu

_E = 320000
_LANES = 128
_ROWS = _E // _LANES


def _concat_body(gen_s, gen_t, gen_w, giv_s, giv_t,
                 out_s, out_t, out_w,
                 ones_buf, sems):
    lo = pl.ds(0, _ROWS)
    hi = pl.ds(_ROWS, _ROWS)
    copies = [
        pltpu.make_async_copy(gen_s.at[lo], out_s.at[lo], sems.at[0]),
        pltpu.make_async_copy(giv_s.at[lo], out_s.at[hi], sems.at[1]),
        pltpu.make_async_copy(gen_t.at[lo], out_t.at[lo], sems.at[2]),
        pltpu.make_async_copy(giv_t.at[lo], out_t.at[hi], sems.at[3]),
        pltpu.make_async_copy(gen_w.at[lo], out_w.at[lo], sems.at[4]),
    ]
    for c in copies:
        c.start()
    ones_buf[...] = jnp.ones((_ROWS, _LANES), jnp.float32)
    ones_copy = pltpu.make_async_copy(ones_buf, out_w.at[hi], sems.at[5])
    ones_copy.start()
    for c in copies:
        c.wait()
    ones_copy.wait()


@jax.jit
def _concat_tc(gen_s, gen_t, gen_w, giv_s, giv_t):
    run = pl.pallas_call(
        _concat_body,
        out_shape=(
            jax.ShapeDtypeStruct((2 * _ROWS, _LANES), jnp.int32),
            jax.ShapeDtypeStruct((2 * _ROWS, _LANES), jnp.int32),
            jax.ShapeDtypeStruct((2 * _ROWS, _LANES), jnp.float32),
        ),
        in_specs=[pl.BlockSpec(memory_space=pl.ANY)] * 5,
        out_specs=(pl.BlockSpec(memory_space=pl.ANY),) * 3,
        scratch_shapes=[
            pltpu.VMEM((_ROWS, _LANES), jnp.float32),
            pltpu.SemaphoreType.DMA((6,)),
        ],
    )
    return run(
        gen_s.reshape(_ROWS, _LANES),
        gen_t.reshape(_ROWS, _LANES),
        gen_w.reshape(_ROWS, _LANES),
        giv_s.reshape(_ROWS, _LANES),
        giv_t.reshape(_ROWS, _LANES),
    )


def kernel(gen_sources, gen_targets, gen_weights, given_sources,
           given_targets, node_embeddings):
    out_s, out_t, out_w = _concat_tc(
        gen_sources, gen_targets, gen_weights, given_sources, given_targets)
    return (out_s.reshape(-1), out_t.reshape(-1), out_w.reshape(-1),
            node_embeddings)


# speedup vs baseline: 4.3912x; 4.3912x over previous
import jax
import jax.numpy as jnp
from jax.experimental import pallas as pl
from jax.experimental.pallas import tpu as pltpu

_E = 320000
_COLS = 625
_ROWS = _E // _COLS
_BLK = 64
_HALF = _ROWS // _BLK
_GRID = 2 * _HALF


def _concat_body(gen_s, gen_t, gen_w, giv_s, giv_t, out_s, out_t, out_w):
    g = pl.program_id(0)

    @pl.when(g < _HALF)
    def _():
        out_s[...] = gen_s[...]
        out_t[...] = gen_t[...]
        out_w[...] = gen_w[...]

    @pl.when(g >= _HALF)
    def _():
        out_s[...] = giv_s[...]
        out_t[...] = giv_t[...]
        out_w[...] = jnp.ones_like(out_w)


@jax.jit
def _concat_tc(gen_s, gen_t, gen_w, giv_s, giv_t):
    gen_map = lambda g: (jnp.minimum(g, _HALF - 1), 0)
    giv_map = lambda g: (jnp.maximum(g - _HALF, 0), 0)
    out_map = lambda g: (g, 0)
    run = pl.pallas_call(
        _concat_body,
        grid=(_GRID,),
        out_shape=(
            jax.ShapeDtypeStruct((2 * _ROWS, _COLS), jnp.int32),
            jax.ShapeDtypeStruct((2 * _ROWS, _COLS), jnp.int32),
            jax.ShapeDtypeStruct((2 * _ROWS, _COLS), jnp.float32),
        ),
        in_specs=[
            pl.BlockSpec((_BLK, _COLS), gen_map),
            pl.BlockSpec((_BLK, _COLS), gen_map),
            pl.BlockSpec((_BLK, _COLS), gen_map),
            pl.BlockSpec((_BLK, _COLS), giv_map),
            pl.BlockSpec((_BLK, _COLS), giv_map),
        ],
        out_specs=(
            pl.BlockSpec((_BLK, _COLS), out_map),
            pl.BlockSpec((_BLK, _COLS), out_map),
            pl.BlockSpec((_BLK, _COLS), out_map),
        ),
    )
    return run(
        gen_s.reshape(_ROWS, _COLS),
        gen_t.reshape(_ROWS, _COLS),
        gen_w.reshape(_ROWS, _COLS),
        giv_s.reshape(_ROWS, _COLS),
        giv_t.reshape(_ROWS, _COLS),
    )


def kernel(gen_sources, gen_targets, gen_weights, given_sources,
           given_targets, node_embeddings):
    out_s, out_t, out_w = _concat_tc(
        gen_sources, gen_targets, gen_weights, given_sources, given_targets)
    return (out_s.reshape(-1), out_t.reshape(-1), out_w.reshape(-1),
            node_embeddings)
